# Initial kernel scaffold; baseline (speedup 1.0000x reference)
import jax
import jax.numpy as jnp
from jax.experimental import pallas as pl


def kernel(code_tensor, word_edge, line_edge, emb, W_gcn_w, b_gcn_w, ln_w_g, ln_w_b, W_att_w, b_att_w, u_w, W_gcn_s, b_gcn_s, ln_s_g, ln_s_b, W_att_s, b_att_s, u_s, W_fc, b_fc):
    raise NotImplementedError("write your pallas kernel here")


# speedup vs baseline: 28.6546x; 28.6546x over previous
import functools

import jax
import jax.numpy as jnp
from jax import lax
from jax.experimental import pallas as pl
from jax.experimental.pallas import tpu as pltpu
from jax.experimental.pallas import tpu_sc as plsc

B, L, S = 8, 128, 50
EMB, HW = 50, 64
N_W = B * L * S
E_W = 819200
N_S = B * L
E_S = 16384
FH = 16
NQ = 4
CH = 128
NCHUNK_W = E_W // CH
NCHUNK_T = N_W // CH
GRP = 8

_sc_mesh = plsc.VectorSubcoreMesh(core_axis_name="c", subcore_axis_name="s")


@functools.partial(
    pl.kernel,
    out_type=(
        jax.ShapeDtypeStruct((N_W, HW), jnp.float32),
        jax.ShapeDtypeStruct((2, N_W), jnp.float32),
        jax.ShapeDtypeStruct((N_S * N_S,), jnp.float32),
    ),
    mesh=_sc_mesh,
    compiler_params=pltpu.CompilerParams(use_tc_tiling_on_sc=False),
    scratch_types=[
        pltpu.VMEM((GRP, CH), jnp.int32),
        pltpu.VMEM((1, CH), jnp.int32),
        pltpu.VMEM((CH, HW), jnp.float32),
        pltpu.VMEM((CH,), jnp.float32),
        pltpu.VMEM_SHARED((N_W,), jnp.float32),
        pltpu.VMEM_SHARED((N_S * N_S,), jnp.float32),
        pltpu.SemaphoreType.DMA,
        pltpu.SemaphoreType.DMA,
    ],
)
def _prep_kernel(toks2_hbm, wdst2_hbm, ladj2_hbm, embp_hbm, zeros_hbm,
                 x_out, degp_out, adj_out,
                 idx_v, tok_v, rows_v, ones_v, degacc, adjacc, sem_g, sem_s):
    c = lax.axis_index("c")
    s = lax.axis_index("s")
    wid = c * 16 + s

    def _fill(i, _):
        ones_v[pl.ds(i * 16, 16)] = jnp.ones((16,), jnp.float32)
        return 0
    lax.fori_loop(0, CH // 16, _fill, 0)

    pltpu.sync_copy(zeros_hbm.at[pl.ds(0, N_W // 16)],
                    degacc.at[pl.ds(s * (N_W // 16), N_W // 16)])

    @pl.when(c == 0)
    def _():
        pltpu.sync_copy(zeros_hbm.at[pl.ds(0, N_S * N_S // 16)],
                        adjacc.at[pl.ds(s * (N_S * N_S // 16), N_S * N_S // 16)])

    plsc.subcore_barrier()

    def _gather_body(k, _):
        chunk = wid + 32 * k

        @pl.when(chunk < NCHUNK_T)
        def _():
            pltpu.sync_copy(toks2_hbm.at[chunk], tok_v.at[0])
            pltpu.async_copy(embp_hbm.at[tok_v.at[0]], rows_v, sem_g).wait()
            pltpu.sync_copy(rows_v, x_out.at[pl.ds(chunk * CH, CH)])
        return 0
    lax.fori_loop(0, (NCHUNK_T + 31) // 32, _gather_body, 0)

    n_tile_chunks = NCHUNK_W // 2 // 16

    def _deg_body(g, _):
        cid0 = c * (NCHUNK_W // 2) + s * n_tile_chunks + g * GRP
        pltpu.sync_copy(wdst2_hbm.at[pl.ds(cid0, GRP)], idx_v)
        descs = []
        for t in range(GRP):
            descs.append(pltpu.async_copy(
                ones_v, degacc.at[idx_v.at[t]], sem_s, add=True))
        for d in descs:
            d.wait()
        return 0
    lax.fori_loop(0, n_tile_chunks // GRP, _deg_body, 0)

    @pl.when(c == 0)
    def _():
        pltpu.sync_copy(ladj2_hbm.at[pl.ds(s * 8, 8)], idx_v)
        descs = []
        for t in range(8):
            descs.append(pltpu.async_copy(
                ones_v, adjacc.at[idx_v.at[t]], sem_s, add=True))
        for d in descs:
            d.wait()

    plsc.subcore_barrier()

    pltpu.sync_copy(degacc.at[pl.ds(s * (N_W // 16), N_W // 16)],
                    degp_out.at[c, pl.ds(s * (N_W // 16), N_W // 16)])

    @pl.when(c == 0)
    def _():
        pltpu.sync_copy(adjacc.at[pl.ds(s * (N_S * N_S // 16), N_S * N_S // 16)],
                        adj_out.at[pl.ds(s * (N_S * N_S // 16), N_S * N_S // 16)])


@functools.partial(
    pl.kernel,
    out_type=jax.ShapeDtypeStruct((NQ, N_W, FH), jnp.float32),
    mesh=_sc_mesh,
    compiler_params=pltpu.CompilerParams(use_tc_tiling_on_sc=False),
    scratch_types=[
        pltpu.VMEM((GRP, CH), jnp.int32),
        pltpu.VMEM((GRP, CH), jnp.int32),
        pltpu.VMEM((GRP, CH, FH), jnp.float32),
        pltpu.VMEM_SHARED((N_W, FH), jnp.float32),
        pltpu.SemaphoreType.DMA,
        pltpu.SemaphoreType.DMA,
    ],
)
def _agg_kernel(ystk_hbm, srcstk_hbm, wdst2_hbm, agg_out,
                src_v, dst_v, rows_v, acc, sem_g, sem_s):
    c = lax.axis_index("c")
    s = lax.axis_index("s")
    rows_per_tile = N_W // 16
    n_tile_chunks = NCHUNK_W // 16

    for q in range(2):
        k = 2 * c + q
        pltpu.sync_copy(
            ystk_hbm.at[pl.ds(k * N_W + s * rows_per_tile, rows_per_tile)],
            acc.at[pl.ds(s * rows_per_tile, rows_per_tile)])
        plsc.subcore_barrier()

        def _body(g, _):
            cid0 = s * n_tile_chunks + g * GRP
            pltpu.sync_copy(srcstk_hbm.at[k, pl.ds(cid0, GRP)], src_v)
            pltpu.sync_copy(wdst2_hbm.at[pl.ds(cid0, GRP)], dst_v)
            gd = []
            for t in range(GRP):
                gd.append(pltpu.async_copy(
                    ystk_hbm.at[src_v.at[t]], rows_v.at[t], sem_g))
            sd = []
            for t in range(GRP):
                gd[t].wait()
                sd.append(pltpu.async_copy(
                    rows_v.at[t], acc.at[dst_v.at[t]], sem_s, add=True))
            for d in sd:
                d.wait()
            return 0
        lax.fori_loop(0, n_tile_chunks // GRP, _body, 0)

        plsc.subcore_barrier()
        pltpu.sync_copy(acc.at[pl.ds(s * rows_per_tile, rows_per_tile)],
                        agg_out.at[k, pl.ds(s * rows_per_tile, rows_per_tile)])


_RB = 6400


def _y_body(x_ref, degp_ref, ystk_ref, dinv_ref):
    deg = degp_ref[0] + degp_ref[1] + 1.0
    dinv = lax.rsqrt(deg)
    y = x_ref[...] * dinv
    for k in range(NQ):
        ystk_ref[k] = y[:, k * FH:(k + 1) * FH]
    dinv_ref[...] = dinv


def _make_y(x, degp):
    grid = N_W // _RB
    return pl.pallas_call(
        _y_body,
        grid=(grid,),
        in_specs=[
            pl.BlockSpec((_RB, HW), lambda i: (i, 0)),
            pl.BlockSpec((2, _RB, 1), lambda i: (0, i, 0)),
        ],
        out_specs=[
            pl.BlockSpec((NQ, _RB, FH), lambda i: (0, i, 0)),
            pl.BlockSpec((_RB, 1), lambda i: (i, 0)),
        ],
        out_shape=[
            jax.ShapeDtypeStruct((NQ, N_W, FH), jnp.float32),
            jax.ShapeDtypeStruct((N_W, 1), jnp.float32),
        ],
    )(x, degp)


_RD = 3200
_NSB = _RD // S


def _word_body(aggp_ref, dinv_ref, wp_ref, b_ref, g_ref, bln_ref,
               watt_ref, bat_ref, u_ref, watt_out, sents_out):
    wp = wp_ref[...]
    aw = sum(jnp.dot(aggp_ref[k], wp[k * FH:(k + 1) * FH, :],
                     preferred_element_type=jnp.float32) for k in range(NQ))
    wo = dinv_ref[...] * aw + b_ref[...]
    mu = jnp.mean(wo, axis=-1, keepdims=True)
    xc = wo - mu
    var = jnp.mean(xc * xc, axis=-1, keepdims=True)
    nrm = xc * lax.rsqrt(var + 1e-5) * g_ref[...] + bln_ref[...]
    t = jnp.tanh(jnp.dot(nrm, watt_ref[...], preferred_element_type=jnp.float32)
                 + bat_ref[...])
    a = jnp.dot(t, u_ref[...], preferred_element_type=jnp.float32)
    a3 = a.reshape(_NSB, S, 1)
    e = jnp.exp(a3 - jnp.max(a3, axis=1, keepdims=True))
    watt = e / jnp.sum(e, axis=1, keepdims=True)
    watt_out[...] = watt
    sents_out[...] = jnp.sum(wo.reshape(_NSB, S, HW) * watt, axis=1)


def _make_word(aggp, dinv, Wp, b, g, bln, Watt, bat, u):
    grid = N_W // _RD
    return pl.pallas_call(
        _word_body,
        grid=(grid,),
        in_specs=[
            pl.BlockSpec((NQ, _RD, FH), lambda i: (0, i, 0)),
            pl.BlockSpec((_RD, 1), lambda i: (i, 0)),
            pl.BlockSpec((HW, HW), lambda i: (0, 0)),
            pl.BlockSpec((1, HW), lambda i: (0, 0)),
            pl.BlockSpec((1, HW), lambda i: (0, 0)),
            pl.BlockSpec((1, HW), lambda i: (0, 0)),
            pl.BlockSpec((HW, HW), lambda i: (0, 0)),
            pl.BlockSpec((1, HW), lambda i: (0, 0)),
            pl.BlockSpec((HW, 1), lambda i: (0, 0)),
        ],
        out_specs=[
            pl.BlockSpec((_NSB, S, 1), lambda i: (i, 0, 0)),
            pl.BlockSpec((_NSB, HW), lambda i: (i, 0)),
        ],
        out_shape=[
            jax.ShapeDtypeStruct((N_S, S, 1), jnp.float32),
            jax.ShapeDtypeStruct((N_S, HW), jnp.float32),
        ],
    )(aggp, dinv, Wp, b, g, bln, Watt, bat, u)


def _line_body(sents_ref, adj_ref, ws_ref, bs_ref, g_ref, bln_ref,
               watt_ref, bat_ref, u_ref, wfc_ref, bfc_ref,
               fin_out, satt_out):
    A = adj_ref[...]
    deg = jnp.sum(A, axis=1, keepdims=True) + 1.0
    dinv = lax.rsqrt(deg)
    sents = sents_ref[...]
    ys = sents * dinv
    h = jnp.dot(A, ys, preferred_element_type=jnp.float32) + ys
    lo = dinv * jnp.dot(h, ws_ref[...], preferred_element_type=jnp.float32) \
        + bs_ref[...]
    mu = jnp.mean(lo, axis=-1, keepdims=True)
    xc = lo - mu
    var = jnp.mean(xc * xc, axis=-1, keepdims=True)
    nrm = xc * lax.rsqrt(var + 1e-5) * g_ref[...] + bln_ref[...]
    t = jnp.tanh(jnp.dot(nrm, watt_ref[...], preferred_element_type=jnp.float32)
                 + bat_ref[...])
    a = jnp.dot(t, u_ref[...], preferred_element_type=jnp.float32)
    a3 = a.reshape(B, L, 1)
    e = jnp.exp(a3 - jnp.max(a3, axis=1, keepdims=True))
    satt = e / jnp.sum(e, axis=1, keepdims=True)
    satt_out[...] = satt
    code = jnp.sum(lo.reshape(B, L, HW) * satt, axis=1)
    fin_out[...] = jax.nn.sigmoid(
        jnp.dot(code, wfc_ref[...], preferred_element_type=jnp.float32)
        + bfc_ref[...])


def _make_line(sents, adj, Ws, bs, g, bln, Watt, bat, u, Wfc, bfc):
    return pl.pallas_call(
        _line_body,
        out_shape=[
            jax.ShapeDtypeStruct((B, 1), jnp.float32),
            jax.ShapeDtypeStruct((B, L, 1), jnp.float32),
        ],
    )(sents, adj, Ws, bs, g, bln, Watt, bat, u, Wfc, bfc)


def kernel(code_tensor, word_edge, line_edge, emb, W_gcn_w, b_gcn_w, ln_w_g,
           ln_w_b, W_att_w, b_att_w, u_w, W_gcn_s, b_gcn_s, ln_s_g, ln_s_b,
           W_att_s, b_att_s, u_s, W_fc, b_fc):
    toks2 = code_tensor.reshape(NCHUNK_T, CH).astype(jnp.int32)
    wsrc = word_edge[0].astype(jnp.int32)
    wdst2 = word_edge[1].astype(jnp.int32).reshape(NCHUNK_W, CH)
    srcstk = jnp.stack([wsrc + k * N_W for k in range(NQ)]).reshape(
        NQ, NCHUNK_W, CH)
    ladj2 = (line_edge[1].astype(jnp.int32) * N_S
             + line_edge[0].astype(jnp.int32)).reshape(E_S // CH, CH)
    embp = jnp.pad(emb, ((0, 0), (0, HW - EMB)))
    Wp = jnp.pad(W_gcn_w, ((0, HW - EMB), (0, 0)))
    zeros = jnp.zeros((N_S * N_S // 16,), jnp.float32)

    x, degp, adj = _prep_kernel(toks2, wdst2, ladj2, embp, zeros)

    ystk, dinv = _make_y(x, degp.reshape(2, N_W, 1))

    aggp = _agg_kernel(ystk.reshape(NQ * N_W, FH), srcstk, wdst2)

    watt, sents = _make_word(
        aggp, dinv, Wp, b_gcn_w.reshape(1, HW), ln_w_g.reshape(1, HW),
        ln_w_b.reshape(1, HW), W_att_w, b_att_w.reshape(1, HW),
        u_w.reshape(HW, 1))

    fin, satt = _make_line(
        sents, adj.reshape(N_S, N_S), W_gcn_s, b_gcn_s.reshape(1, HW),
        ln_s_g.reshape(1, HW), ln_s_b.reshape(1, HW), W_att_s,
        b_att_s.reshape(1, HW), u_s.reshape(HW, 1), W_fc,
        b_fc.reshape(1, 1))

    return (fin, watt.reshape(B, L, S), satt.reshape(B, L),
            sents.reshape(B, L, HW))


# speedup vs baseline: 33.9976x; 1.1865x over previous
import functools

import jax
import jax.numpy as jnp
from jax import lax
from jax.experimental import pallas as pl
from jax.experimental.pallas import tpu as pltpu
from jax.experimental.pallas import tpu_sc as plsc

B, L, S = 8, 128, 50
EMB, HW = 50, 64
N_W = B * L * S
E_W = 819200
N_S = B * L
E_S = 16384
FH = 16
NQ = 4
CH = 128
NCHUNK_W = E_W // CH
NCHUNK_T = N_W // CH
GRP = 10

_sc_mesh = plsc.VectorSubcoreMesh(core_axis_name="c", subcore_axis_name="s")


@functools.partial(
    pl.kernel,
    out_type=(
        jax.ShapeDtypeStruct((N_W, HW), jnp.float32),
        jax.ShapeDtypeStruct((2, N_W), jnp.float32),
        jax.ShapeDtypeStruct((N_S * N_S,), jnp.float32),
    ),
    mesh=_sc_mesh,
    compiler_params=pltpu.CompilerParams(use_tc_tiling_on_sc=False),
    scratch_types=[
        pltpu.VMEM((GRP, CH), jnp.int32),
        pltpu.VMEM((1, CH), jnp.int32),
        pltpu.VMEM((CH, HW), jnp.float32),
        pltpu.VMEM((CH,), jnp.float32),
        pltpu.VMEM_SHARED((N_W,), jnp.float32),
        pltpu.VMEM_SHARED((N_S * N_S,), jnp.float32),
        pltpu.SemaphoreType.DMA,
        pltpu.SemaphoreType.DMA,
    ],
)
def _prep_kernel(toks2_hbm, wdst2_hbm, ladj2_hbm, embp_hbm, zeros_hbm,
                 x_out, degp_out, adj_out,
                 idx_v, tok_v, rows_v, ones_v, degacc, adjacc, sem_g, sem_s):
    c = lax.axis_index("c")
    s = lax.axis_index("s")
    wid = c * 16 + s

    def _fill(i, _):
        ones_v[pl.ds(i * 16, 16)] = jnp.ones((16,), jnp.float32)
        return 0
    lax.fori_loop(0, CH // 16, _fill, 0)

    pltpu.sync_copy(zeros_hbm.at[pl.ds(0, N_W // 16)],
                    degacc.at[pl.ds(s * (N_W // 16), N_W // 16)])

    @pl.when(c == 0)
    def _():
        pltpu.sync_copy(zeros_hbm.at[pl.ds(0, N_S * N_S // 16)],
                        adjacc.at[pl.ds(s * (N_S * N_S // 16), N_S * N_S // 16)])

    plsc.subcore_barrier()

    def _gather_body(k, _):
        chunk = wid + 32 * k

        @pl.when(chunk < NCHUNK_T)
        def _():
            pltpu.sync_copy(toks2_hbm.at[chunk], tok_v.at[0])
            pltpu.async_copy(embp_hbm.at[tok_v.at[0]], rows_v, sem_g).wait()
            pltpu.sync_copy(rows_v, x_out.at[pl.ds(chunk * CH, CH)])
        return 0
    lax.fori_loop(0, (NCHUNK_T + 31) // 32, _gather_body, 0)

    n_tile_chunks = NCHUNK_W // 2 // 16

    def _deg_body(g, _):
        cid0 = c * (NCHUNK_W // 2) + s * n_tile_chunks + g * GRP
        pltpu.sync_copy(wdst2_hbm.at[pl.ds(cid0, GRP)], idx_v)
        descs = []
        for t in range(GRP):
            descs.append(pltpu.async_copy(
                ones_v, degacc.at[idx_v.at[t]], sem_s, add=True))
        for d in descs:
            d.wait()
        return 0
    lax.fori_loop(0, n_tile_chunks // GRP, _deg_body, 0)

    @pl.when(c == 0)
    def _():
        pltpu.sync_copy(ladj2_hbm.at[pl.ds(s * 8, 8)], idx_v.at[pl.ds(0, 8)])
        descs = []
        for t in range(8):
            descs.append(pltpu.async_copy(
                ones_v, adjacc.at[idx_v.at[t]], sem_s, add=True))
        for d in descs:
            d.wait()

    plsc.subcore_barrier()

    pltpu.sync_copy(degacc.at[pl.ds(s * (N_W // 16), N_W // 16)],
                    degp_out.at[c, pl.ds(s * (N_W // 16), N_W // 16)])

    @pl.when(c == 0)
    def _():
        pltpu.sync_copy(adjacc.at[pl.ds(s * (N_S * N_S // 16), N_S * N_S // 16)],
                        adj_out.at[pl.ds(s * (N_S * N_S // 16), N_S * N_S // 16)])


_MAC = 50
_NGM = _MAC // GRP


@functools.partial(
    pl.kernel,
    out_type=jax.ShapeDtypeStruct((NQ, N_W, FH), jnp.float32),
    mesh=_sc_mesh,
    compiler_params=pltpu.CompilerParams(use_tc_tiling_on_sc=False),
    scratch_types=[
        pltpu.VMEM((_MAC, 2, CH), jnp.int32),
        pltpu.VMEM((2, GRP, CH, FH), jnp.float32),
        pltpu.VMEM_SHARED((N_W, FH), jnp.float32),
        pltpu.SemaphoreType.DMA,
        pltpu.SemaphoreType.DMA,
        pltpu.SemaphoreType.DMA,
    ],
)
def _agg_kernel(ystk_hbm, catstk_hbm, agg_out,
                idx_v, rows_v, acc, sem_g0, sem_g1, sem_s):
    c = lax.axis_index("c")
    s = lax.axis_index("s")
    rows_per_tile = N_W // 16
    n_tile_chunks = NCHUNK_W // 16
    sems = (sem_g0, sem_g1)

    def _fire(g, buf, sem):
        ds = []
        for t in range(GRP):
            ds.append(pltpu.async_copy(
                ystk_hbm.at[idx_v.at[g * GRP + t, 0]], rows_v.at[buf, t], sem))
        return ds

    def _wait(g, buf, sem):
        for t in range(GRP):
            pltpu.make_async_copy(
                ystk_hbm.at[idx_v.at[g * GRP + t, 0]],
                rows_v.at[buf, t], sem).wait()

    def _scatter(g, buf):
        sd = []
        for t in range(GRP):
            sd.append(pltpu.async_copy(
                rows_v.at[buf, t], acc.at[idx_v.at[g * GRP + t, 1]],
                sem_s, add=True))
        for d in sd:
            d.wait()

    for q in range(2):
        k = 2 * c + q
        pltpu.sync_copy(
            ystk_hbm.at[pl.ds(k * N_W + s * rows_per_tile, rows_per_tile)],
            acc.at[pl.ds(s * rows_per_tile, rows_per_tile)])
        plsc.subcore_barrier()

        for half in range(n_tile_chunks // _MAC):
            pltpu.sync_copy(
                catstk_hbm.at[k, pl.ds(s * n_tile_chunks + half * _MAC, _MAC)],
                idx_v)
            _fire(0, 0, sem_g0)

            def _pair(p, _):
                g0 = 2 * p
                _fire(g0 + 1, 1, sem_g1)
                _wait(g0, 0, sem_g0)
                _scatter(g0, 0)
                _fire(g0 + 2, 0, sem_g0)
                _wait(g0 + 1, 1, sem_g1)
                _scatter(g0 + 1, 1)
                return 0
            lax.fori_loop(0, (_NGM - 1) // 2, _pair, 0)
            _wait(_NGM - 1, 0, sem_g0)
            _scatter(_NGM - 1, 0)

        plsc.subcore_barrier()
        pltpu.sync_copy(acc.at[pl.ds(s * rows_per_tile, rows_per_tile)],
                        agg_out.at[k, pl.ds(s * rows_per_tile, rows_per_tile)])


_RB = 6400


def _y_body(x_ref, degp_ref, ystk_ref, dinv_ref):
    deg = degp_ref[0] + degp_ref[1] + 1.0
    dinv = lax.rsqrt(deg)
    y = x_ref[...] * dinv
    for k in range(NQ):
        ystk_ref[k] = y[:, k * FH:(k + 1) * FH]
    dinv_ref[...] = dinv


def _make_y(x, degp):
    grid = N_W // _RB
    return pl.pallas_call(
        _y_body,
        grid=(grid,),
        in_specs=[
            pl.BlockSpec((_RB, HW), lambda i: (i, 0)),
            pl.BlockSpec((2, _RB, 1), lambda i: (0, i, 0)),
        ],
        out_specs=[
            pl.BlockSpec((NQ, _RB, FH), lambda i: (0, i, 0)),
            pl.BlockSpec((_RB, 1), lambda i: (i, 0)),
        ],
        out_shape=[
            jax.ShapeDtypeStruct((NQ, N_W, FH), jnp.float32),
            jax.ShapeDtypeStruct((N_W, 1), jnp.float32),
        ],
    )(x, degp)


_RD = 3200
_NSB = _RD // S


def _word_body(aggp_ref, dinv_ref, wp_ref, b_ref, g_ref, bln_ref,
               watt_ref, bat_ref, u_ref, watt_out, sents_out):
    wp = wp_ref[...]
    aw = sum(jnp.dot(aggp_ref[k], wp[k * FH:(k + 1) * FH, :],
                     preferred_element_type=jnp.float32) for k in range(NQ))
    wo = dinv_ref[...] * aw + b_ref[...]
    mu = jnp.mean(wo, axis=-1, keepdims=True)
    xc = wo - mu
    var = jnp.mean(xc * xc, axis=-1, keepdims=True)
    nrm = xc * lax.rsqrt(var + 1e-5) * g_ref[...] + bln_ref[...]
    t = jnp.tanh(jnp.dot(nrm, watt_ref[...], preferred_element_type=jnp.float32)
                 + bat_ref[...])
    a = jnp.dot(t, u_ref[...], preferred_element_type=jnp.float32)
    a3 = a.reshape(_NSB, S, 1)
    e = jnp.exp(a3 - jnp.max(a3, axis=1, keepdims=True))
    watt = e / jnp.sum(e, axis=1, keepdims=True)
    watt_out[...] = watt
    sents_out[...] = jnp.sum(wo.reshape(_NSB, S, HW) * watt, axis=1)


def _make_word(aggp, dinv, Wp, b, g, bln, Watt, bat, u):
    grid = N_W // _RD
    return pl.pallas_call(
        _word_body,
        grid=(grid,),
        in_specs=[
            pl.BlockSpec((NQ, _RD, FH), lambda i: (0, i, 0)),
            pl.BlockSpec((_RD, 1), lambda i: (i, 0)),
            pl.BlockSpec((HW, HW), lambda i: (0, 0)),
            pl.BlockSpec((1, HW), lambda i: (0, 0)),
            pl.BlockSpec((1, HW), lambda i: (0, 0)),
            pl.BlockSpec((1, HW), lambda i: (0, 0)),
            pl.BlockSpec((HW, HW), lambda i: (0, 0)),
            pl.BlockSpec((1, HW), lambda i: (0, 0)),
            pl.BlockSpec((HW, 1), lambda i: (0, 0)),
        ],
        out_specs=[
            pl.BlockSpec((_NSB, S, 1), lambda i: (i, 0, 0)),
            pl.BlockSpec((_NSB, HW), lambda i: (i, 0)),
        ],
        out_shape=[
            jax.ShapeDtypeStruct((N_S, S, 1), jnp.float32),
            jax.ShapeDtypeStruct((N_S, HW), jnp.float32),
        ],
    )(aggp, dinv, Wp, b, g, bln, Watt, bat, u)


def _line_body(sents_ref, adj_ref, ws_ref, bs_ref, g_ref, bln_ref,
               watt_ref, bat_ref, u_ref, wfc_ref, bfc_ref,
               fin_out, satt_out):
    A = adj_ref[...]
    deg = jnp.sum(A, axis=1, keepdims=True) + 1.0
    dinv = lax.rsqrt(deg)
    sents = sents_ref[...]
    ys = sents * dinv
    h = jnp.dot(A, ys, preferred_element_type=jnp.float32) + ys
    lo = dinv * jnp.dot(h, ws_ref[...], preferred_element_type=jnp.float32) \
        + bs_ref[...]
    mu = jnp.mean(lo, axis=-1, keepdims=True)
    xc = lo - mu
    var = jnp.mean(xc * xc, axis=-1, keepdims=True)
    nrm = xc * lax.rsqrt(var + 1e-5) * g_ref[...] + bln_ref[...]
    t = jnp.tanh(jnp.dot(nrm, watt_ref[...], preferred_element_type=jnp.float32)
                 + bat_ref[...])
    a = jnp.dot(t, u_ref[...], preferred_element_type=jnp.float32)
    a3 = a.reshape(B, L, 1)
    e = jnp.exp(a3 - jnp.max(a3, axis=1, keepdims=True))
    satt = e / jnp.sum(e, axis=1, keepdims=True)
    satt_out[...] = satt
    code = jnp.sum(lo.reshape(B, L, HW) * satt, axis=1)
    fin_out[...] = jax.nn.sigmoid(
        jnp.dot(code, wfc_ref[...], preferred_element_type=jnp.float32)
        + bfc_ref[...])


def _make_line(sents, adj, Ws, bs, g, bln, Watt, bat, u, Wfc, bfc):
    return pl.pallas_call(
        _line_body,
        out_shape=[
            jax.ShapeDtypeStruct((B, 1), jnp.float32),
            jax.ShapeDtypeStruct((B, L, 1), jnp.float32),
        ],
    )(sents, adj, Ws, bs, g, bln, Watt, bat, u, Wfc, bfc)


def kernel(code_tensor, word_edge, line_edge, emb, W_gcn_w, b_gcn_w, ln_w_g,
           ln_w_b, W_att_w, b_att_w, u_w, W_gcn_s, b_gcn_s, ln_s_g, ln_s_b,
           W_att_s, b_att_s, u_s, W_fc, b_fc):
    toks2 = code_tensor.reshape(NCHUNK_T, CH).astype(jnp.int32)
    wsrc = word_edge[0].astype(jnp.int32)
    wdst2 = word_edge[1].astype(jnp.int32).reshape(NCHUNK_W, CH)
    srcstk = jnp.stack([wsrc + k * N_W for k in range(NQ)]).reshape(
        NQ, NCHUNK_W, CH)
    catstk = jnp.stack(
        [srcstk, jnp.broadcast_to(wdst2[None], (NQ, NCHUNK_W, CH))], axis=2)
    ladj2 = (line_edge[1].astype(jnp.int32) * N_S
             + line_edge[0].astype(jnp.int32)).reshape(E_S // CH, CH)
    embp = jnp.pad(emb, ((0, 0), (0, HW - EMB)))
    Wp = jnp.pad(W_gcn_w, ((0, HW - EMB), (0, 0)))
    zeros = jnp.zeros((N_S * N_S // 16,), jnp.float32)

    x, degp, adj = _prep_kernel(toks2, wdst2, ladj2, embp, zeros)

    ystk, dinv = _make_y(x, degp.reshape(2, N_W, 1))

    aggp = _agg_kernel(ystk.reshape(NQ * N_W, FH), catstk)

    watt, sents = _make_word(
        aggp, dinv, Wp, b_gcn_w.reshape(1, HW), ln_w_g.reshape(1, HW),
        ln_w_b.reshape(1, HW), W_att_w, b_att_w.reshape(1, HW),
        u_w.reshape(HW, 1))

    fin, satt = _make_line(
        sents, adj.reshape(N_S, N_S), W_gcn_s, b_gcn_s.reshape(1, HW),
        ln_s_g.reshape(1, HW), ln_s_b.reshape(1, HW), W_att_s,
        b_att_s.reshape(1, HW), u_s.reshape(HW, 1), W_fc,
        b_fc.reshape(1, 1))

    return (fin, watt.reshape(B, L, S), satt.reshape(B, L),
            sents.reshape(B, L, HW))
